# baseline (speedup 1.0000x reference)
import jax
import jax.numpy as jnp
from jax.experimental import pallas as pl
from jax.experimental.pallas import tpu as pltpu

N = 256
R = 7
D = 768
H = 150
NREL = 6
WB = 3
CSTEPS = 2 * (NREL // WB)
BI = 64
STEPS = CSTEPS + N // BI


def _fused_kernel(x_ref, labpadT_ref, root1_ref, bias1_ref, root2_ref,
                  bias2_ref, w1_ref, w2_ref, w1_pair_ref, b1_ref,
                  wm2_ref, b2_ref, wm3_ref, b3_ref, out_ref,
                  h_s, o_s, v_s, m_s, mean_s):
    s = pl.program_id(0)
    bf = jnp.bfloat16

    @pl.when(s == 0)
    def _build_masks():
        lt = labpadT_ref[...]
        shifted = jnp.concatenate(
            [jnp.full((1, N), 6, jnp.int32), lt[: N - 1, :]], axis=0
        )
        kk = jax.lax.broadcasted_iota(jnp.int32, (N, N), 0)
        ii = jax.lax.broadcasted_iota(jnp.int32, (N, N), 1)
        labT = jnp.where(kk < ii, lt, jnp.where(kk > ii, shifted, 6))
        for r in range(NREL):
            mr = (labT == r).astype(jnp.float32)
            cnt = jnp.sum(mr, axis=1, keepdims=True)
            m_s[pl.ds(r * N, N), :] = (mr / jnp.maximum(cnt, 1.0)).astype(bf)

    half = jax.lax.rem(s, 2)

    def _conv_step(xb, root_ref, bias_ref, w_ref, acc_ref, first):
        if first:
            mean_s[...] = jnp.dot(
                m_s[...], xb, preferred_element_type=jnp.float32
            ).astype(bf)
        rbase = 0 if first else WB
        contrib = None
        for j in range(WB):
            c = jnp.dot(mean_s[pl.ds((rbase + j) * N, N), :],
                        w_ref[j].astype(bf),
                        preferred_element_type=jnp.float32)
            contrib = c if contrib is None else contrib + c
        if first:
            base = jnp.dot(xb, root_ref[...].astype(bf),
                           preferred_element_type=jnp.float32)
            acc_ref[...] = base + bias_ref[...] + contrib
        else:
            acc_ref[...] = acc_ref[...] + contrib

    @pl.when(s == 0)
    def _c1_first():
        _conv_step(x_ref[...].astype(bf), root1_ref, bias1_ref, w1_ref,
                   h_s, True)

    @pl.when(s == 1)
    def _c1_rest():
        _conv_step(None, root1_ref, bias1_ref, w1_ref, h_s, False)
        h_s[...] = jnp.maximum(h_s[...], 0.0)

    @pl.when(s == 2)
    def _c2_first():
        _conv_step(h_s[...].astype(bf), root2_ref, bias2_ref, w2_ref,
                   o_s, True)

    @pl.when(s == 3)
    def _c2_rest():
        _conv_step(None, root2_ref, bias2_ref, w2_ref, o_s, False)

    @pl.when(s >= CSTEPS)
    def _pair():
        i = s - CSTEPS

        @pl.when(i == 0)
        def _precompute_v():
            v = jnp.dot(
                o_s[...].astype(bf), w1_pair_ref[D:, :].astype(bf),
                preferred_element_type=jnp.float32,
            ).astype(bf)
            v_s[...] = v.T

        xb = o_s[pl.ds(i * BI, BI), :].astype(bf)
        u = jnp.dot(xb, w1_pair_ref[:D, :].astype(bf),
                    preferred_element_type=jnp.float32)
        ut = (u.T + b1_ref[...]).astype(bf)
        h1 = jnp.maximum(
            ut[:, :, None] + v_s[...][:, None, :],
            jnp.asarray(0.0, bf),
        )
        h2 = jnp.maximum(
            jax.lax.dot_general(wm2_ref[...].astype(bf), h1, (((0,), (0,)), ((), ())),
                                preferred_element_type=jnp.float32)
            + b2_ref[...][:, :, None],
            0.0,
        )
        sc = jax.lax.dot_general(wm3_ref[...].astype(bf), h2.astype(bf),
                                 (((0,), (0,)), ((), ())),
                                 preferred_element_type=jnp.float32)
        sc = sc + b3_ref[...][:, :, None]
        ig = i * BI + jax.lax.broadcasted_iota(jnp.int32, (1, BI, N - 1), 1)
        jj = jax.lax.broadcasted_iota(jnp.int32, (1, BI, N - 1), 2)
        out_ref[...] = jnp.where(jj < ig, sc[:, :, : N - 1], sc[:, :, 1:])


def _full(shape):
    return pl.BlockSpec(shape, lambda s: (0,) * len(shape))


def kernel(event_embed, labels, bW1, bb1, bW2, bb2, bW3, bb3,
           cW1, cb1, cW2, cb2, cW3, cb3,
           W1, root1, bias1, W2, root2, bias2):
    x = event_embed[0]
    labpadT = jnp.concatenate(
        [labels.reshape(N, N - 1), jnp.full((N, 1), 6, jnp.int32)], axis=1
    ).T
    scores = pl.pallas_call(
        _fused_kernel,
        grid=(STEPS,),
        in_specs=[
            _full((N, D)),
            _full((N, N)),
            _full((D, D)),
            _full((1, D)),
            _full((D, D)),
            _full((1, D)),
            pl.BlockSpec((WB, D, D), lambda s: (jnp.clip(s, 0, 1), 0, 0)),
            pl.BlockSpec((WB, D, D), lambda s: (jnp.clip(s - 2, 0, 1), 0, 0)),
            _full((2 * D, H)),
            _full((H, 1)),
            _full((H, H)),
            _full((H, 1)),
            _full((H, R)),
            _full((R, 1)),
        ],
        out_specs=pl.BlockSpec(
            (R, BI, N - 1), lambda s: (0, jnp.clip(s - CSTEPS, 0, N // BI - 1), 0)
        ),
        out_shape=jax.ShapeDtypeStruct((R, N, N - 1), jnp.float32),
        scratch_shapes=[
            pltpu.VMEM((N, D), jnp.float32),
            pltpu.VMEM((N, D), jnp.float32),
            pltpu.VMEM((H, N), jnp.bfloat16),
            pltpu.VMEM((NREL * N, N), jnp.bfloat16),
            pltpu.VMEM((NREL * N, D), jnp.bfloat16),
        ],
    )(x, labpadT, root1, bias1.reshape(1, D), root2, bias2.reshape(1, D),
      W1, W2, cW1, cb1.reshape(H, 1), cW2, cb2.reshape(H, 1),
      cW3, cb3.reshape(R, 1))
    return scores.transpose(1, 2, 0).reshape(1, N * (N - 1), R)


# speedup vs baseline: 1.3327x; 1.3327x over previous
import jax
import jax.numpy as jnp
from jax.experimental import pallas as pl
from jax.experimental.pallas import tpu as pltpu

N = 256
R = 7
D = 768
H = 150
NREL = 6
WB = 3
CSTEPS = 2 * (NREL // WB)
BI = 64
STEPS = CSTEPS + N // BI


def _fused_kernel(x_ref, labpadT_ref, root1_ref, bias1_ref, root2_ref,
                  bias2_ref, w1_ref, w2_ref, w1_pair_ref, b1_ref,
                  wm2_ref, b2_ref, wm3_ref, b3_ref, out_ref,
                  h_s, o_s, v_s, m_s, mean_s):
    s = pl.program_id(0)
    bf = jnp.bfloat16

    @pl.when(s == 0)
    def _build_masks():
        lt = labpadT_ref[...]
        shifted = jnp.concatenate(
            [jnp.full((1, N), 6, jnp.int32), lt[: N - 1, :]], axis=0
        )
        kk = jax.lax.broadcasted_iota(jnp.int32, (N, N), 0)
        ii = jax.lax.broadcasted_iota(jnp.int32, (N, N), 1)
        labT = jnp.where(kk < ii, lt, jnp.where(kk > ii, shifted, 6))
        for r in range(NREL):
            mr = (labT == r).astype(jnp.float32)
            cnt = jnp.sum(mr, axis=1, keepdims=True)
            m_s[pl.ds(r * N, N), :] = (mr / jnp.maximum(cnt, 1.0)).astype(bf)

    half = jax.lax.rem(s, 2)

    def _conv_step(xb, root_ref, bias_ref, w_ref, acc_ref, first):
        if first:
            mean_s[...] = jnp.dot(
                m_s[...], xb, preferred_element_type=jnp.float32
            ).astype(bf)
        rbase = 0 if first else WB
        contrib = None
        for j in range(WB):
            c = jnp.dot(mean_s[pl.ds((rbase + j) * N, N), :],
                        w_ref[j].astype(bf),
                        preferred_element_type=jnp.float32)
            contrib = c if contrib is None else contrib + c
        if first:
            base = jnp.dot(xb, root_ref[...].astype(bf),
                           preferred_element_type=jnp.float32)
            acc_ref[...] = base + bias_ref[...] + contrib
        else:
            acc_ref[...] = acc_ref[...] + contrib

    @pl.when(s == 0)
    def _c1_first():
        _conv_step(x_ref[...].astype(bf), root1_ref, bias1_ref, w1_ref,
                   h_s, True)

    @pl.when(s == 1)
    def _c1_rest():
        _conv_step(None, root1_ref, bias1_ref, w1_ref, h_s, False)
        h_s[...] = jnp.maximum(h_s[...], 0.0)

    @pl.when(s == 2)
    def _c2_first():
        _conv_step(h_s[...].astype(bf), root2_ref, bias2_ref, w2_ref,
                   o_s, True)

    @pl.when(s == 3)
    def _c2_rest():
        _conv_step(None, root2_ref, bias2_ref, w2_ref, o_s, False)

    @pl.when(s >= CSTEPS)
    def _pair():
        i = s - CSTEPS

        @pl.when(i == 0)
        def _precompute_v():
            v = jnp.dot(
                o_s[...].astype(bf), w1_pair_ref[D:, :].astype(bf),
                preferred_element_type=jnp.float32,
            ).astype(bf)
            v_s[...] = v.T

        xb = o_s[pl.ds(i * BI, BI), :].astype(bf)
        u = jnp.dot(xb, w1_pair_ref[:D, :].astype(bf),
                    preferred_element_type=jnp.float32)
        ut = (u.T + b1_ref[...]).astype(bf)
        h1 = jnp.maximum(
            ut[:, :, None] + v_s[...][:, None, :],
            jnp.asarray(0.0, bf),
        )
        h1 = h1.reshape(H, BI * N)
        h2 = jnp.maximum(
            jax.lax.dot_general(wm2_ref[...].astype(bf), h1, (((0,), (0,)), ((), ())),
                                preferred_element_type=jnp.float32)
            + b2_ref[...],
            0.0,
        )
        sc = jax.lax.dot_general(wm3_ref[...].astype(bf), h2.astype(bf),
                                 (((0,), (0,)), ((), ())),
                                 preferred_element_type=jnp.float32)
        sc = (sc + b3_ref[...]).astype(bf)
        sc = sc.reshape(R, BI, N)
        ig = i * BI + jax.lax.broadcasted_iota(jnp.int32, (1, BI, N - 1), 1)
        jj = jax.lax.broadcasted_iota(jnp.int32, (1, BI, N - 1), 2)
        out_ref[...] = jnp.where(jj < ig, sc[:, :, : N - 1], sc[:, :, 1:])


def _full(shape):
    return pl.BlockSpec(shape, lambda s: (0,) * len(shape))


def kernel(event_embed, labels, bW1, bb1, bW2, bb2, bW3, bb3,
           cW1, cb1, cW2, cb2, cW3, cb3,
           W1, root1, bias1, W2, root2, bias2):
    x = event_embed[0]
    labpadT = jnp.concatenate(
        [labels.reshape(N, N - 1), jnp.full((N, 1), 6, jnp.int32)], axis=1
    ).T
    scores = pl.pallas_call(
        _fused_kernel,
        grid=(STEPS,),
        in_specs=[
            _full((N, D)),
            _full((N, N)),
            _full((D, D)),
            _full((1, D)),
            _full((D, D)),
            _full((1, D)),
            pl.BlockSpec((WB, D, D), lambda s: (jnp.clip(s, 0, 1), 0, 0)),
            pl.BlockSpec((WB, D, D), lambda s: (jnp.clip(s - 2, 0, 1), 0, 0)),
            _full((2 * D, H)),
            _full((H, 1)),
            _full((H, H)),
            _full((H, 1)),
            _full((H, R)),
            _full((R, 1)),
        ],
        out_specs=pl.BlockSpec(
            (R, BI, N - 1), lambda s: (0, jnp.clip(s - CSTEPS, 0, N // BI - 1), 0)
        ),
        out_shape=jax.ShapeDtypeStruct((R, N, N - 1), jnp.bfloat16),
        scratch_shapes=[
            pltpu.VMEM((N, D), jnp.float32),
            pltpu.VMEM((N, D), jnp.float32),
            pltpu.VMEM((H, N), jnp.bfloat16),
            pltpu.VMEM((NREL * N, N), jnp.bfloat16),
            pltpu.VMEM((NREL * N, D), jnp.bfloat16),
        ],
    )(x, labpadT, root1, bias1.reshape(1, D), root2, bias2.reshape(1, D),
      W1, W2, cW1, cb1.reshape(H, 1), cW2, cb2.reshape(H, 1),
      cW3, cb3.reshape(R, 1))
    return scores.transpose(1, 2, 0).astype(jnp.float32).reshape(1, N * (N - 1), R)


# speedup vs baseline: 1.3360x; 1.0025x over previous
import jax
import jax.numpy as jnp
from jax.experimental import pallas as pl
from jax.experimental.pallas import tpu as pltpu

N = 256
R = 7
D = 768
H = 150
NREL = 6
WB = 3
CSTEPS = 2 * (NREL // WB)
BI = 64
STEPS = CSTEPS + N // BI


def _fused_kernel(x_ref, labpadT_ref, root1_ref, bias1_ref, root2_ref,
                  bias2_ref, w1_ref, w2_ref, w1_pair_ref, b1_ref,
                  wm2_ref, b2_ref, wm3_ref, b3_ref, out_ref,
                  h_s, o_s, v_s, m_s, mean_s):
    s = pl.program_id(0)
    bf = jnp.bfloat16

    @pl.when(s == 0)
    def _build_masks():
        lt = labpadT_ref[...]
        shifted = jnp.concatenate(
            [jnp.full((1, N), 6, jnp.int32), lt[: N - 1, :]], axis=0
        )
        kk = jax.lax.broadcasted_iota(jnp.int32, (N, N), 0)
        ii = jax.lax.broadcasted_iota(jnp.int32, (N, N), 1)
        labT = jnp.where(kk < ii, lt, jnp.where(kk > ii, shifted, 6))
        for r in range(NREL):
            mr = (labT == r).astype(jnp.float32)
            cnt = jnp.sum(mr, axis=1, keepdims=True)
            m_s[pl.ds(r * N, N), :] = (mr / jnp.maximum(cnt, 1.0)).astype(bf)

    half = jax.lax.rem(s, 2)

    def _conv_step(xb, root_ref, bias_ref, w_ref, acc_ref, first):
        if first:
            mean_s[...] = jnp.dot(
                m_s[...], xb, preferred_element_type=jnp.float32
            ).astype(bf)
        rbase = 0 if first else WB
        contrib = None
        for j in range(WB):
            c = jnp.dot(mean_s[pl.ds((rbase + j) * N, N), :],
                        w_ref[j].astype(bf),
                        preferred_element_type=jnp.float32)
            contrib = c if contrib is None else contrib + c
        if first:
            base = jnp.dot(xb, root_ref[...].astype(bf),
                           preferred_element_type=jnp.float32)
            acc_ref[...] = base + bias_ref[...] + contrib
        else:
            acc_ref[...] = acc_ref[...] + contrib

    @pl.when(s == 0)
    def _c1_first():
        _conv_step(x_ref[...].astype(bf), root1_ref, bias1_ref, w1_ref,
                   h_s, True)

    @pl.when(s == 1)
    def _c1_rest():
        _conv_step(None, root1_ref, bias1_ref, w1_ref, h_s, False)
        h_s[...] = jnp.maximum(h_s[...], 0.0)

    @pl.when(s == 2)
    def _c2_first():
        _conv_step(h_s[...].astype(bf), root2_ref, bias2_ref, w2_ref,
                   o_s, True)

    @pl.when(s == 3)
    def _c2_rest():
        _conv_step(None, root2_ref, bias2_ref, w2_ref, o_s, False)

    @pl.when(s >= CSTEPS)
    def _pair():
        i = s - CSTEPS

        @pl.when(i == 0)
        def _precompute_v():
            v = jnp.dot(
                o_s[...].astype(bf), w1_pair_ref[D:, :].astype(bf),
                preferred_element_type=jnp.float32,
            ).astype(bf)
            v_s[...] = v.T

        xb = o_s[pl.ds(i * BI, BI), :].astype(bf)
        u = jnp.dot(xb, w1_pair_ref[:D, :].astype(bf),
                    preferred_element_type=jnp.float32)
        ut = (u.T + b1_ref[...]).astype(bf)
        h1 = jnp.maximum(
            ut[:, :, None] + v_s[...][:, None, :],
            jnp.asarray(0.0, bf),
        )
        h1 = h1.reshape(H, BI * N)
        h2 = jnp.maximum(
            jax.lax.dot_general(wm2_ref[...].astype(bf), h1, (((0,), (0,)), ((), ())),
                                preferred_element_type=jnp.float32)
            + b2_ref[...],
            0.0,
        )
        sc = jax.lax.dot_general(wm3_ref[...], h2,
                                 (((0,), (0,)), ((), ())),
                                 preferred_element_type=jnp.float32)
        sc = (sc + b3_ref[...]).astype(bf)
        sc = sc.reshape(R, BI, N)
        ig = i * BI + jax.lax.broadcasted_iota(jnp.int32, (1, BI, N - 1), 1)
        jj = jax.lax.broadcasted_iota(jnp.int32, (1, BI, N - 1), 2)
        out_ref[...] = jnp.where(jj < ig, sc[:, :, : N - 1], sc[:, :, 1:])


def _full(shape):
    return pl.BlockSpec(shape, lambda s: (0,) * len(shape))


def kernel(event_embed, labels, bW1, bb1, bW2, bb2, bW3, bb3,
           cW1, cb1, cW2, cb2, cW3, cb3,
           W1, root1, bias1, W2, root2, bias2):
    x = event_embed[0]
    labpadT = jnp.concatenate(
        [labels.reshape(N, N - 1), jnp.full((N, 1), 6, jnp.int32)], axis=1
    ).T
    scores = pl.pallas_call(
        _fused_kernel,
        grid=(STEPS,),
        in_specs=[
            _full((N, D)),
            _full((N, N)),
            _full((D, D)),
            _full((1, D)),
            _full((D, D)),
            _full((1, D)),
            pl.BlockSpec((WB, D, D), lambda s: (jnp.clip(s, 0, 1), 0, 0)),
            pl.BlockSpec((WB, D, D), lambda s: (jnp.clip(s - 2, 0, 1), 0, 0)),
            _full((2 * D, H)),
            _full((H, 1)),
            _full((H, H)),
            _full((H, 1)),
            _full((H, R)),
            _full((R, 1)),
        ],
        out_specs=pl.BlockSpec(
            (R, BI, N - 1), lambda s: (0, jnp.clip(s - CSTEPS, 0, N // BI - 1), 0)
        ),
        out_shape=jax.ShapeDtypeStruct((R, N, N - 1), jnp.bfloat16),
        scratch_shapes=[
            pltpu.VMEM((N, D), jnp.float32),
            pltpu.VMEM((N, D), jnp.float32),
            pltpu.VMEM((H, N), jnp.bfloat16),
            pltpu.VMEM((NREL * N, N), jnp.bfloat16),
            pltpu.VMEM((NREL * N, D), jnp.bfloat16),
        ],
    )(x, labpadT, root1, bias1.reshape(1, D), root2, bias2.reshape(1, D),
      W1, W2, cW1, cb1.reshape(H, 1), cW2, cb2.reshape(H, 1),
      cW3, cb3.reshape(R, 1))
    return scores.transpose(1, 2, 0).astype(jnp.float32).reshape(1, N * (N - 1), R)


# speedup vs baseline: 1.3935x; 1.0430x over previous
import jax
import jax.numpy as jnp
from jax.experimental import pallas as pl
from jax.experimental.pallas import tpu as pltpu

N = 256
R = 7
D = 768
H = 150
NREL = 6
WB = 3
CSTEPS = 2 * (NREL // WB)
BI = 64
STEPS = CSTEPS + N // BI


def _fused_kernel(x_ref, labpadT_ref, root1_ref, bias1_ref, root2_ref,
                  bias2_ref, w1_ref, w2_ref, w1_pair_ref, b1_ref,
                  wm2_ref, b2_ref, wm3_ref, b3_ref, out_ref,
                  h_s, o_s, v_s, m_s, mean_s, w_buf, w_sem):
    s = pl.program_id(0)
    bf = jnp.bfloat16

    def _w_dma(c):
        src_ref = (w1_ref, w1_ref, w2_ref, w2_ref)[c]
        lo = (0, WB, 0, WB)[c]
        return pltpu.make_async_copy(
            src_ref.at[pl.ds(lo, WB)], w_buf.at[c], w_sem.at[c]
        )

    @pl.when(s == 0)
    def _start_w_dmas():
        for c in range(4):
            _w_dma(c).start()

    @pl.when(s == 0)
    def _build_masks():
        lt = labpadT_ref[...]
        shifted = jnp.concatenate(
            [jnp.full((1, N), 6, jnp.int32), lt[: N - 1, :]], axis=0
        )
        kk = jax.lax.broadcasted_iota(jnp.int32, (N, N), 0)
        ii = jax.lax.broadcasted_iota(jnp.int32, (N, N), 1)
        labT = jnp.where(kk < ii, lt, jnp.where(kk > ii, shifted, 6))
        for r in range(NREL):
            mr = (labT == r).astype(jnp.float32)
            cnt = jnp.sum(mr, axis=1, keepdims=True)
            m_s[pl.ds(r * N, N), :] = (mr / jnp.maximum(cnt, 1.0)).astype(bf)

    half = jax.lax.rem(s, 2)

    def _conv_step(xb, root_ref, bias_ref, chunk, acc_ref, first):
        if first:
            mean_s[...] = jnp.dot(
                m_s[...], xb, preferred_element_type=jnp.float32
            ).astype(bf)
        rbase = 0 if first else WB
        _w_dma(chunk).wait()
        contrib = None
        for j in range(WB):
            c = jnp.dot(mean_s[pl.ds((rbase + j) * N, N), :],
                        w_buf[chunk, j].astype(bf),
                        preferred_element_type=jnp.float32)
            contrib = c if contrib is None else contrib + c
        if first:
            base = jnp.dot(xb, root_ref[...].astype(bf),
                           preferred_element_type=jnp.float32)
            acc_ref[...] = base + bias_ref[...] + contrib
        else:
            acc_ref[...] = acc_ref[...] + contrib

    @pl.when(s == 0)
    def _c1_first():
        _conv_step(x_ref[...].astype(bf), root1_ref, bias1_ref, 0,
                   h_s, True)

    @pl.when(s == 1)
    def _c1_rest():
        _conv_step(None, root1_ref, bias1_ref, 1, h_s, False)
        h_s[...] = jnp.maximum(h_s[...], 0.0)

    @pl.when(s == 2)
    def _c2_first():
        _conv_step(h_s[...].astype(bf), root2_ref, bias2_ref, 2,
                   o_s, True)

    @pl.when(s == 3)
    def _c2_rest():
        _conv_step(None, root2_ref, bias2_ref, 3, o_s, False)

    @pl.when(s >= CSTEPS)
    def _pair():
        i = s - CSTEPS

        @pl.when(i == 0)
        def _precompute_v():
            v = jnp.dot(
                o_s[...].astype(bf), w1_pair_ref[D:, :].astype(bf),
                preferred_element_type=jnp.float32,
            ).astype(bf)
            v_s[...] = v.T

        xb = o_s[pl.ds(i * BI, BI), :].astype(bf)
        u = jnp.dot(xb, w1_pair_ref[:D, :].astype(bf),
                    preferred_element_type=jnp.float32)
        ut = (u.T + b1_ref[...]).astype(bf)
        h1 = jnp.maximum(
            ut[:, :, None] + v_s[...][:, None, :],
            jnp.asarray(0.0, bf),
        )
        h1 = h1.reshape(H, BI * N)
        h2 = jnp.maximum(
            jax.lax.dot_general(wm2_ref[...].astype(bf), h1, (((0,), (0,)), ((), ())),
                                preferred_element_type=jnp.float32)
            + b2_ref[...],
            0.0,
        )
        sc = jax.lax.dot_general(wm3_ref[...], h2,
                                 (((0,), (0,)), ((), ())),
                                 preferred_element_type=jnp.float32)
        sc = (sc + b3_ref[...]).astype(bf)
        sc = sc.reshape(R, BI, N)
        ig = i * BI + jax.lax.broadcasted_iota(jnp.int32, (1, BI, N - 1), 1)
        jj = jax.lax.broadcasted_iota(jnp.int32, (1, BI, N - 1), 2)
        out_ref[...] = jnp.where(jj < ig, sc[:, :, : N - 1], sc[:, :, 1:])


def _full(shape):
    return pl.BlockSpec(shape, lambda s: (0,) * len(shape))


def kernel(event_embed, labels, bW1, bb1, bW2, bb2, bW3, bb3,
           cW1, cb1, cW2, cb2, cW3, cb3,
           W1, root1, bias1, W2, root2, bias2):
    x = event_embed[0]
    labpadT = jnp.concatenate(
        [labels.reshape(N, N - 1), jnp.full((N, 1), 6, jnp.int32)], axis=1
    ).T
    scores = pl.pallas_call(
        _fused_kernel,
        grid=(STEPS,),
        in_specs=[
            _full((N, D)),
            _full((N, N)),
            _full((D, D)),
            _full((1, D)),
            _full((D, D)),
            _full((1, D)),
            pl.BlockSpec(memory_space=pl.ANY),
            pl.BlockSpec(memory_space=pl.ANY),
            _full((2 * D, H)),
            _full((H, 1)),
            _full((H, H)),
            _full((H, 1)),
            _full((H, R)),
            _full((R, 1)),
        ],
        out_specs=pl.BlockSpec(
            (R, BI, N - 1), lambda s: (0, jnp.clip(s - CSTEPS, 0, N // BI - 1), 0)
        ),
        out_shape=jax.ShapeDtypeStruct((R, N, N - 1), jnp.bfloat16),
        scratch_shapes=[
            pltpu.VMEM((N, D), jnp.float32),
            pltpu.VMEM((N, D), jnp.float32),
            pltpu.VMEM((H, N), jnp.bfloat16),
            pltpu.VMEM((NREL * N, N), jnp.bfloat16),
            pltpu.VMEM((NREL * N, D), jnp.bfloat16),
            pltpu.VMEM((4, WB, D, D), jnp.float32),
            pltpu.SemaphoreType.DMA((4,)),
        ],
    )(x, labpadT, root1, bias1.reshape(1, D), root2, bias2.reshape(1, D),
      W1, W2, cW1, cb1.reshape(H, 1), cW2, cb2.reshape(H, 1),
      cW3, cb3.reshape(R, 1))
    return scores.transpose(1, 2, 0).astype(jnp.float32).reshape(1, N * (N - 1), R)
